# baseline (speedup 1.0000x reference)
import functools

import jax
import jax.numpy as jnp
from jax import lax
from jax.experimental import pallas as pl
from jax.experimental.pallas import tpu as pltpu
from jax.experimental.pallas import tpu_sc as plsc

_V = 462
_E = 462
_D = 256
_L = 16
_EPW = 32
_NW = 16
_W = 2 * _EPW + 4

_mesh = plsc.VectorSubcoreMesh(core_axis_name="c", subcore_axis_name="s",
                               num_cores=1)


@functools.partial(
    pl.kernel,
    mesh=_mesh,
    out_type=jax.ShapeDtypeStruct((2 * _V, _D), jnp.float32),
    scratch_types=[
        pltpu.VMEM((_W,), jnp.int32),
        pltpu.VMEM((2 * _EPW,), jnp.int32),
        pltpu.VMEM((_EPW,), jnp.int32),
        pltpu.VMEM((2 * _EPW, _D), jnp.float32),
        pltpu.VMEM((_EPW, _D), jnp.float32),
        pltpu.VMEM((_EPW, _D), jnp.float32),
        pltpu.SemaphoreType.DMA,
        pltpu.SemaphoreType.DMA,
        pltpu.SemaphoreType.DMA,
    ],
)
def _gunpool_sc(x_hbm, pidx_hbm, out_hbm, win_v, idx_v, crow_v, rows_v,
                mid_v, copy_v, sem0, sem1, sem2):
    w = lax.axis_index("s")
    base = jnp.minimum(w * _EPW, _E - _EPW)
    lane = lax.broadcasted_iota(jnp.int32, (_L,), 0)

    start = jnp.minimum(w * (2 * _EPW), 2 * _E - _W)
    off = 2 * base - start
    wf = pltpu.async_copy(pidx_hbm.at[pl.ds(start, _W)], win_v, sem1)

    for h in range(_EPW // _L):
        crow_v[pl.ds(h * _L, _L)] = base + h * _L + lane
    cp_in = pltpu.async_copy(x_hbm.at[crow_v], copy_v, sem2)

    wf.wait()
    for h in range(2 * _EPW // _L):
        idx_v[pl.ds(h * _L, _L)] = win_v[pl.ds(off + h * _L, _L)]
    g = pltpu.async_copy(x_hbm.at[idx_v], rows_v, sem0)

    cp_in.wait()
    cp_out = pltpu.async_copy(copy_v, out_hbm.at[crow_v], sem2)

    g.wait()
    def _avg_row(i, carry):
        for j in range(_D // _L):
            s = pl.ds(j * _L, _L)
            mid_v[i, s] = 0.5 * (rows_v[2 * i, s] + rows_v[2 * i + 1, s])
        return carry

    lax.fori_loop(0, _EPW, _avg_row, 0, unroll=False)

    cp_out.wait()
    for h in range(_EPW // _L):
        crow_v[pl.ds(h * _L, _L)] = base + h * _L + lane + _V
    scat = pltpu.async_copy(mid_v, out_hbm.at[crow_v], sem0)
    scat.wait()


def kernel(input, pool_idx):
    return _gunpool_sc(input, pool_idx.reshape(-1))
